# baseline (speedup 1.0000x reference)
import numpy as np
import jax
import jax.numpy as jnp
from jax.experimental import pallas as pl
from jax.experimental.pallas import tpu as pltpu

_MIN_GAP = 4
_MAX_GAP = 48
_REPEATED_SAMPLING = 4


def _temporal_indices(t: int):
    max_gap = min(_MAX_GAP, t - 1)
    gap = np.linspace(_MIN_GAP, max_gap, _REPEATED_SAMPLING).astype(np.int32)
    return [0] + [int(g) for g in gap]


def _copy_body(x_ref, o_ref):
    o_ref[...] = x_ref[...]


def kernel(x):
    b, c, t, h, w = x.shape
    idx = _temporal_indices(t)
    k = len(idx)
    bc = b * c
    xv = x.reshape(bc, t, h, w)

    def _dma_body(x_ref, o_ref, sems):
        copies = [
            pltpu.make_async_copy(x_ref.at[:, src], o_ref.at[:, j], sems.at[j])
            for j, src in enumerate(idx)
        ]
        for cp in copies:
            cp.start()
        for cp in copies:
            cp.wait()

    out = pl.pallas_call(
        _dma_body,
        in_specs=[pl.BlockSpec(memory_space=pl.ANY)],
        out_specs=pl.BlockSpec(memory_space=pl.ANY),
        out_shape=jax.ShapeDtypeStruct((bc, k, h, w), x.dtype),
        scratch_shapes=[pltpu.SemaphoreType.DMA((k,))],
    )(xv)
    return out.reshape(b, c, k, h, w)


# speedup vs baseline: 2.7628x; 2.7628x over previous
import functools

import numpy as np
import jax
import jax.numpy as jnp
from jax import lax
from jax.experimental import pallas as pl
from jax.experimental.pallas import tpu as pltpu
from jax.experimental.pallas import tpu_sc as plsc

_MIN_GAP = 4
_MAX_GAP = 48
_REPEATED_SAMPLING = 4


def _temporal_indices(t: int):
    max_gap = min(_MAX_GAP, t - 1)
    gap = np.linspace(_MIN_GAP, max_gap, _REPEATED_SAMPLING).astype(np.int32)
    return [0] + [int(g) for g in gap]


def kernel(x):
    b, c, t, h, w = x.shape
    idx = _temporal_indices(t)
    k = len(idx)
    bc = b * c
    hw = h * w
    n_planes = bc * k

    info = plsc.get_sparse_core_info()
    nc, ns = info.num_cores, info.num_subcores
    nw = nc * ns

    spp = 8
    ch = hw // spp
    n_units = n_planes * spp
    per_w = n_units // nw
    nbuf = min(4, per_w)

    xflat = x.reshape(bc * t * hw)

    mesh = plsc.VectorSubcoreMesh(core_axis_name="c", subcore_axis_name="s")
    scratch = [pltpu.VMEM((ch,), x.dtype) for _ in range(nbuf)] + [
        pltpu.SemaphoreType.DMA for _ in range(2 * nbuf)
    ]

    @functools.partial(
        pl.kernel,
        mesh=mesh,
        out_type=jax.ShapeDtypeStruct((n_planes * hw,), x.dtype),
        scratch_types=scratch,
    )
    def sc_copy(x_hbm, o_hbm, *sc):
        bufs = sc[:nbuf]
        gsem = sc[nbuf:2 * nbuf]
        ssem = sc[2 * nbuf:3 * nbuf]
        wid = lax.axis_index("s") * nc + lax.axis_index("c")

        def offs(i):
            u = wid + nw * i
            p = u // spp
            s = u - p * spp
            r = p % k
            tsel = sum(v * (r == kk) for kk, v in enumerate(idx))
            src = ((p // k) * t + tsel) * hw + s * ch
            dst = u * ch
            return src, dst

        gh = [None] * nbuf
        sh = [None] * nbuf
        dst_of = [None] * nbuf
        for i in range(per_w):
            bi = i % nbuf
            if i >= nbuf:
                sh[bi].wait()
            src, dst = offs(i)
            dst_of[bi] = dst
            gh[bi] = pltpu.async_copy(
                x_hbm.at[pl.ds(src, ch)], bufs[bi], gsem[bi])
            if i >= 1:
                pb = (i - 1) % nbuf
                gh[pb].wait()
                sh[pb] = pltpu.async_copy(
                    bufs[pb], o_hbm.at[pl.ds(dst_of[pb], ch)], ssem[pb])
        lb = (per_w - 1) % nbuf
        gh[lb].wait()
        sh[lb] = pltpu.async_copy(
            bufs[lb], o_hbm.at[pl.ds(dst_of[lb], ch)], ssem[lb])
        for bi in range(min(nbuf, per_w)):
            sh[bi].wait()

    out = sc_copy(xflat)
    return out.reshape(b, c, k, h, w)


# speedup vs baseline: 14.4300x; 5.2229x over previous
import functools

import numpy as np
import jax
import jax.numpy as jnp
from jax import lax
from jax.experimental import pallas as pl
from jax.experimental.pallas import tpu as pltpu
from jax.experimental.pallas import tpu_sc as plsc

_MIN_GAP = 4
_MAX_GAP = 48
_REPEATED_SAMPLING = 4


def _temporal_indices(t: int):
    max_gap = min(_MAX_GAP, t - 1)
    gap = np.linspace(_MIN_GAP, max_gap, _REPEATED_SAMPLING).astype(np.int32)
    return [0] + [int(g) for g in gap]


def kernel(x):
    b, c, t, h, w = x.shape
    idx = _temporal_indices(t)
    k = len(idx)
    bc = b * c
    n_planes = bc * k
    n_pairs = n_planes // 2

    info = plsc.get_sparse_core_info()
    nc, ns = info.num_cores, info.num_subcores

    xv = x.reshape(bc * t, h, w)

    mesh = plsc.VectorSubcoreMesh(core_axis_name="c", subcore_axis_name="s")
    scratch = [
        pltpu.VMEM((h, w), x.dtype),
        pltpu.VMEM((h, w), x.dtype),
        pltpu.SemaphoreType.DMA,
        pltpu.SemaphoreType.DMA,
        pltpu.SemaphoreType.DMA,
        pltpu.SemaphoreType.DMA,
    ]

    @functools.partial(
        pl.kernel,
        mesh=mesh,
        out_type=jax.ShapeDtypeStruct((n_planes, h, w), x.dtype),
        scratch_types=scratch,
    )
    def sc_copy(x_hbm, o_hbm, buf0, buf1, gs0, gs1, ss0, ss1):
        wid = lax.axis_index("s") * nc + lax.axis_index("c")

        def src_plane(u):
            p = u // k
            r = u - p * k
            tsel = sum(v * (r == kk) for kk, v in enumerate(idx))
            return p * t + tsel

        u0 = 2 * wid
        u1 = 2 * wid + 1

        @pl.when(wid < n_pairs)
        def _():
            g0 = pltpu.async_copy(x_hbm.at[src_plane(u0)], buf0, gs0)
            g1 = pltpu.async_copy(x_hbm.at[src_plane(u1)], buf1, gs1)
            g0.wait()
            s0 = pltpu.async_copy(buf0, o_hbm.at[u0], ss0)
            g1.wait()
            s1 = pltpu.async_copy(buf1, o_hbm.at[u1], ss1)
            s0.wait()
            s1.wait()

    out = sc_copy(xv)
    return out.reshape(b, c, k, h, w)
